# baseline (speedup 1.0000x reference)
import jax
import jax.numpy as jnp
from jax import lax
from jax.experimental import pallas as pl
from jax.experimental.pallas import tpu as pltpu
from jax.experimental.pallas import tpu_sc as plsc

_NC = 2
_NS = 16
_K = 80
_EPS = 1e-5


def _sc_scatter(y, src1, dst1, zeros):
    N, D = y.shape
    E = src1.shape[0]
    ept = E // (_NC * _NS)
    NCH = ept // _K
    Np = zeros.shape[0]
    RB = Np // _NS

    mesh = plsc.VectorSubcoreMesh(core_axis_name="c", subcore_axis_name="s")
    out_type = jax.ShapeDtypeStruct((_NC, Np, D), jnp.float32)
    scratch = [
        pltpu.VMEM((2, _K), jnp.int32),
        pltpu.VMEM((2, _K), jnp.int32),
        pltpu.VMEM((_K,), jnp.int32),
        pltpu.VMEM((_K,), jnp.int32),
        pltpu.VMEM((_K,), jnp.int32),
        pltpu.VMEM((4, _K, D), jnp.float32),
        pltpu.VMEM_SHARED((Np, D), jnp.float32),
        pltpu.SemaphoreType.DMA((2,)),
        pltpu.SemaphoreType.DMA((2,)),
        pltpu.SemaphoreType.DMA((4,)),
        pltpu.SemaphoreType.DMA((3,)),
    ]

    def body(y_hbm, src_hbm, dst_hbm, z_hbm, sout, srcv, dstv, dsc0, dsc1,
             dsc2, rows, accum, isem_s, isem_d, gsem, ssem):
        cid = lax.axis_index("c")
        sid = lax.axis_index("s")
        widx = cid * _NS + sid
        r0 = sid * RB
        last = NCH - 1

        def issue_idx(g):
            gc = jnp.minimum(g, last)
            e0 = widx * ept + gc * _K
            b = lax.rem(g, 2)
            pltpu.async_copy(src_hbm.at[pl.ds(e0, _K)], srcv.at[b],
                             isem_s.at[b])
            pltpu.async_copy(dst_hbm.at[pl.ds(e0, _K)], dstv.at[b],
                             isem_d.at[b])

        def wait_idx(g):
            b = lax.rem(g, 2)
            pltpu.make_async_copy(src_hbm.at[pl.ds(0, _K)], srcv.at[b],
                                  isem_s.at[b]).wait()
            pltpu.make_async_copy(dst_hbm.at[pl.ds(0, _K)], dstv.at[b],
                                  isem_d.at[b]).wait()

        def issue_gather(g):
            bi = lax.rem(g, 2)
            br = lax.rem(g, 4)
            pltpu.async_copy(y_hbm.at[srcv.at[bi]], rows.at[br], gsem.at[br])

        def wait_gather(g):
            bi = lax.rem(g, 2)
            br = lax.rem(g, 4)
            pltpu.make_async_copy(y_hbm.at[srcv.at[bi]], rows.at[br],
                                  gsem.at[br]).wait()

        def wait_scatter(g):
            bs = lax.rem(g, 3)
            br = lax.rem(g, 4)
            pltpu.make_async_copy(rows.at[br], accum.at[dsc0],
                                  ssem.at[bs]).wait()

        def issue_scatter(g, dscX):
            b = lax.rem(g, 2)
            bs = lax.rem(g, 3)
            br = lax.rem(g, 4)
            for j in range(_K // 16):
                dscX[pl.ds(j * 16, 16)] = dstv[b, pl.ds(j * 16, 16)]
            pltpu.async_copy(rows.at[br], accum.at[dscX], ssem.at[bs],
                             add=True)

        pltpu.sync_copy(z_hbm.at[pl.ds(r0, RB), :], accum.at[pl.ds(r0, RB), :])
        plsc.subcore_barrier()

        issue_idx(0)
        wait_idx(0)
        issue_gather(0)
        issue_idx(1)

        def step(g, carry):
            bs = lax.rem(g, 3)
            wait_idx(g + 1)

            @pl.when(g >= 3)
            def _():
                wait_scatter(g - 3)
            issue_gather(g + 1)
            wait_gather(g)

            @pl.when(bs == 0)
            def _():
                issue_scatter(g, dsc0)

            @pl.when(bs == 1)
            def _():
                issue_scatter(g, dsc1)

            @pl.when(bs == 2)
            def _():
                issue_scatter(g, dsc2)
            issue_idx(g + 2)
            return carry
        lax.fori_loop(0, NCH, step, 0)

        wait_idx(NCH + 1)
        wait_gather(NCH)
        wait_scatter(NCH - 3)
        wait_scatter(NCH - 2)
        wait_scatter(NCH - 1)

        plsc.subcore_barrier()
        pltpu.sync_copy(accum.at[pl.ds(r0, RB), :],
                        sout.at[cid, pl.ds(r0, RB), :])

    k = pl.kernel(body, out_type=out_type, mesh=mesh, scratch_types=scratch)
    return k(y, src1, dst1, zeros)


def _sc_degree(dst1, zeros, ones):
    E = dst1.shape[0]
    ept = E // (_NC * _NS)
    NCH = ept // _K
    Np, D = zeros.shape
    RB = Np // _NS

    mesh = plsc.VectorSubcoreMesh(core_axis_name="c", subcore_axis_name="s")
    out_type = jax.ShapeDtypeStruct((_NC, Np, D), jnp.float32)
    scratch = [
        pltpu.VMEM((2, _K), jnp.int32),
        pltpu.VMEM((_K,), jnp.int32),
        pltpu.VMEM((_K, D), jnp.float32),
        pltpu.VMEM_SHARED((Np, D), jnp.float32),
        pltpu.SemaphoreType.DMA((2,)),
    ]

    def body(dst_hbm, z_hbm, ones_hbm, dout, dstv, dsc, onesv, dega, isem):
        cid = lax.axis_index("c")
        sid = lax.axis_index("s")
        widx = cid * _NS + sid
        r0 = sid * RB
        last = NCH - 1

        def issue_idx(g):
            gc = jnp.minimum(g, last)
            e0 = widx * ept + gc * _K
            b = lax.rem(g, 2)
            pltpu.async_copy(dst_hbm.at[pl.ds(e0, _K)], dstv.at[b],
                             isem.at[b])

        def wait_idx(g):
            b = lax.rem(g, 2)
            pltpu.make_async_copy(dst_hbm.at[pl.ds(0, _K)], dstv.at[b],
                                  isem.at[b]).wait()

        pltpu.sync_copy(z_hbm.at[pl.ds(r0, RB), :], dega.at[pl.ds(r0, RB), :])
        pltpu.sync_copy(ones_hbm, onesv)
        plsc.subcore_barrier()

        issue_idx(0)

        def step(g, carry):
            b = lax.rem(g, 2)
            wait_idx(g)
            issue_idx(g + 1)
            for j in range(_K // 16):
                dsc[pl.ds(j * 16, 16)] = dstv[b, pl.ds(j * 16, 16)]
            pltpu.sync_copy(onesv, dega.at[dsc], add=True)
            return carry
        lax.fori_loop(0, NCH, step, 0)

        wait_idx(NCH)

        plsc.subcore_barrier()
        pltpu.sync_copy(dega.at[pl.ds(r0, RB), :],
                        dout.at[cid, pl.ds(r0, RB), :])

    k = pl.kernel(body, out_type=out_type, mesh=mesh, scratch_types=scratch)
    return k(dst1, zeros, ones)


def _dotT(a, w):
    return lax.dot_general(a, w, (((1,), (1,)), ((), ())),
                           preferred_element_type=jnp.float32)


def _pre_body(x_ref, wl_ref, wr_ref, b_ref, y_ref, z_ref):
    x = x_ref[...]
    y_ref[...] = _dotT(x, wl_ref[...])
    z_ref[...] = _dotT(x, wr_ref[...]) + b_ref[...]


def _bn_relu(s_ref, degp_ref, z_ref, g_ref, be_ref):
    n = z_ref.shape[0]
    s = (s_ref[0] + s_ref[1])[:n]
    deg = (degp_ref[0] + degp_ref[1])[:n]
    h = s / jnp.maximum(deg, 1.0) + z_ref[...]
    mu = jnp.mean(h, axis=0, keepdims=True)
    ctr = h - mu
    var = jnp.mean(ctr * ctr, axis=0, keepdims=True)
    hn = g_ref[...] * ctr * lax.rsqrt(var + _EPS) + be_ref[...]
    return jnp.maximum(hn, 0.0)


def _mid_body(s_ref, degp_ref, z_ref, g_ref, be_ref, wl_ref, wr_ref, b_ref,
              y2_ref, z2_ref):
    h1 = _bn_relu(s_ref, degp_ref, z_ref, g_ref, be_ref)
    y2_ref[...] = _dotT(h1, wl_ref[...])
    z2_ref[...] = _dotT(h1, wr_ref[...]) + b_ref[...]


def _post_body(s_ref, degp_ref, z_ref, g_ref, be_ref, out_ref):
    out_ref[...] = _bn_relu(s_ref, degp_ref, z_ref, g_ref, be_ref)


def kernel(x, edge_index, Wl1, Wr1, b1, g1, be1, Wl2, Wr2, b2, g2, be2):
    N, D = x.shape
    Np = (N + 16 * 8 - 1) // (16 * 8) * (16 * 8)
    src1 = edge_index[0]
    dst1 = edge_index[1]
    zeros = jnp.zeros((Np, D), jnp.float32)
    ones = jnp.ones((_K, D), jnp.float32)
    f32 = jnp.float32
    sd = jax.ShapeDtypeStruct

    degp = _sc_degree(dst1, zeros, ones)

    y1, z1 = pl.pallas_call(
        _pre_body,
        out_shape=[sd((N, D), f32), sd((N, D), f32)],
    )(x, Wl1, Wr1, b1.reshape(1, D))

    s1 = _sc_scatter(y1, src1, dst1, zeros)

    y2, z2 = pl.pallas_call(
        _mid_body,
        out_shape=[sd((N, D), f32), sd((N, D), f32)],
    )(s1, degp, z1, g1.reshape(1, D), be1.reshape(1, D), Wl2, Wr2,
      b2.reshape(1, D))

    s2 = _sc_scatter(y2, src1, dst1, zeros)

    out = pl.pallas_call(
        _post_body,
        out_shape=sd((N, D), f32),
    )(s2, degp, z2, g2.reshape(1, D), be2.reshape(1, D))

    return out


# speedup vs baseline: 1.0106x; 1.0106x over previous
import jax
import jax.numpy as jnp
from jax import lax
from jax.experimental import pallas as pl
from jax.experimental.pallas import tpu as pltpu
from jax.experimental.pallas import tpu_sc as plsc

_NC = 2
_NS = 16
_K = 80
_EPS = 1e-5


def _sc_scatter(y, src1, dst1, zeros):
    N, D = y.shape
    E = src1.shape[0]
    ept = E // (_NC * _NS)
    NCH = ept // _K
    Np = zeros.shape[0]
    RB = Np // _NS

    mesh = plsc.VectorSubcoreMesh(core_axis_name="c", subcore_axis_name="s")
    out_type = jax.ShapeDtypeStruct((_NC, Np, D), jnp.float32)
    scratch = [
        pltpu.VMEM((2, _K), jnp.int32),
        pltpu.VMEM((2, _K), jnp.int32),
        pltpu.VMEM((_K,), jnp.int32),
        pltpu.VMEM((_K,), jnp.int32),
        pltpu.VMEM((_K,), jnp.int32),
        pltpu.VMEM((4, _K, D), jnp.float32),
        pltpu.VMEM_SHARED((Np, D), jnp.float32),
        pltpu.SemaphoreType.DMA((2,)),
        pltpu.SemaphoreType.DMA((2,)),
        pltpu.SemaphoreType.DMA((4,)),
        pltpu.SemaphoreType.DMA((3,)),
    ]

    def body(y_hbm, src_hbm, dst_hbm, z_hbm, sout, srcv, dstv, dsc0, dsc1,
             dsc2, rows, accum, isem_s, isem_d, gsem, ssem):
        cid = lax.axis_index("c")
        sid = lax.axis_index("s")
        widx = cid * _NS + sid
        r0 = sid * RB
        last = NCH - 1

        def issue_idx(g):
            gc = jnp.minimum(g, last)
            e0 = widx * ept + gc * _K
            b = lax.rem(g, 2)
            pltpu.async_copy(src_hbm.at[pl.ds(e0, _K)], srcv.at[b],
                             isem_s.at[b])
            pltpu.async_copy(dst_hbm.at[pl.ds(e0, _K)], dstv.at[b],
                             isem_d.at[b])

        def wait_idx(g):
            b = lax.rem(g, 2)
            pltpu.make_async_copy(src_hbm.at[pl.ds(0, _K)], srcv.at[b],
                                  isem_s.at[b]).wait()
            pltpu.make_async_copy(dst_hbm.at[pl.ds(0, _K)], dstv.at[b],
                                  isem_d.at[b]).wait()

        def issue_gather(g):
            bi = lax.rem(g, 2)
            br = lax.rem(g, 4)
            pltpu.async_copy(y_hbm.at[srcv.at[bi]], rows.at[br], gsem.at[br])

        def wait_gather(g):
            bi = lax.rem(g, 2)
            br = lax.rem(g, 4)
            pltpu.make_async_copy(y_hbm.at[srcv.at[bi]], rows.at[br],
                                  gsem.at[br]).wait()

        def wait_scatter(g):
            bs = lax.rem(g, 3)
            br = lax.rem(g, 4)
            pltpu.make_async_copy(rows.at[br], accum.at[dsc0],
                                  ssem.at[bs]).wait()

        def issue_scatter(g, dscX):
            b = lax.rem(g, 2)
            bs = lax.rem(g, 3)
            br = lax.rem(g, 4)
            for j in range(_K // 16):
                dscX[pl.ds(j * 16, 16)] = dstv[b, pl.ds(j * 16, 16)]
            pltpu.async_copy(rows.at[br], accum.at[dscX], ssem.at[bs],
                             add=True)

        pltpu.sync_copy(z_hbm.at[pl.ds(r0, RB), :], accum.at[pl.ds(r0, RB), :])
        plsc.subcore_barrier()

        issue_idx(0)
        wait_idx(0)
        issue_gather(0)
        issue_idx(1)

        def step(g, carry):
            bs = lax.rem(g, 3)
            wait_idx(g + 1)

            @pl.when(g >= 3)
            def _():
                wait_scatter(g - 3)
            issue_gather(g + 1)
            wait_gather(g)

            @pl.when(bs == 0)
            def _():
                issue_scatter(g, dsc0)

            @pl.when(bs == 1)
            def _():
                issue_scatter(g, dsc1)

            @pl.when(bs == 2)
            def _():
                issue_scatter(g, dsc2)
            issue_idx(g + 2)
            return carry
        lax.fori_loop(0, NCH, step, 0)

        wait_idx(NCH + 1)
        wait_gather(NCH)
        wait_scatter(NCH - 3)
        wait_scatter(NCH - 2)
        wait_scatter(NCH - 1)

        plsc.subcore_barrier()
        pltpu.sync_copy(accum.at[pl.ds(r0, RB), :],
                        sout.at[cid, pl.ds(r0, RB), :])

    k = pl.kernel(body, out_type=out_type, mesh=mesh, scratch_types=scratch)
    return k(y, src1, dst1, zeros)


def _sc_degree(dst1, zeros, ones):
    E = dst1.shape[0]
    ept = E // (_NC * _NS)
    NCH = ept // _K
    Np, D = zeros.shape
    RB = Np // _NS

    mesh = plsc.VectorSubcoreMesh(core_axis_name="c", subcore_axis_name="s")
    out_type = jax.ShapeDtypeStruct((_NC, Np, D), jnp.float32)
    scratch = [
        pltpu.VMEM((2, _K), jnp.int32),
        pltpu.VMEM((_K,), jnp.int32),
        pltpu.VMEM((_K,), jnp.int32),
        pltpu.VMEM((_K, D), jnp.float32),
        pltpu.VMEM_SHARED((Np, D), jnp.float32),
        pltpu.SemaphoreType.DMA((2,)),
        pltpu.SemaphoreType.DMA((2,)),
    ]

    def body(dst_hbm, z_hbm, ones_hbm, dout, dstv, dsc0, dsc1, onesv, dega,
             isem, ssem):
        cid = lax.axis_index("c")
        sid = lax.axis_index("s")
        widx = cid * _NS + sid
        r0 = sid * RB
        last = NCH - 1

        def issue_idx(g):
            gc = jnp.minimum(g, last)
            e0 = widx * ept + gc * _K
            b = lax.rem(g, 2)
            pltpu.async_copy(dst_hbm.at[pl.ds(e0, _K)], dstv.at[b],
                             isem.at[b])

        def wait_idx(g):
            b = lax.rem(g, 2)
            pltpu.make_async_copy(dst_hbm.at[pl.ds(0, _K)], dstv.at[b],
                                  isem.at[b]).wait()

        def issue_scatter(g, dscX):
            b = lax.rem(g, 2)
            for j in range(_K // 16):
                dscX[pl.ds(j * 16, 16)] = dstv[b, pl.ds(j * 16, 16)]
            pltpu.async_copy(onesv, dega.at[dscX], ssem.at[b], add=True)

        def wait_scatter(g):
            b = lax.rem(g, 2)
            pltpu.make_async_copy(onesv, dega.at[dsc0], ssem.at[b]).wait()

        pltpu.sync_copy(z_hbm.at[pl.ds(r0, RB), :], dega.at[pl.ds(r0, RB), :])
        pltpu.sync_copy(ones_hbm, onesv)
        plsc.subcore_barrier()

        issue_idx(0)

        def step(g, carry):
            b = lax.rem(g, 2)
            wait_idx(g)
            issue_idx(g + 1)

            @pl.when(g >= 2)
            def _():
                wait_scatter(g - 2)

            @pl.when(b == 0)
            def _():
                issue_scatter(g, dsc0)

            @pl.when(b == 1)
            def _():
                issue_scatter(g, dsc1)
            return carry
        lax.fori_loop(0, NCH, step, 0)

        wait_idx(NCH)
        wait_scatter(NCH - 2)
        wait_scatter(NCH - 1)

        plsc.subcore_barrier()
        pltpu.sync_copy(dega.at[pl.ds(r0, RB), :],
                        dout.at[cid, pl.ds(r0, RB), :])

    k = pl.kernel(body, out_type=out_type, mesh=mesh, scratch_types=scratch)
    return k(dst1, zeros, ones)


def _dotT(a, w):
    return lax.dot_general(a, w, (((1,), (1,)), ((), ())),
                           preferred_element_type=jnp.float32)


def _pre_body(x_ref, wl_ref, wr_ref, b_ref, y_ref, z_ref):
    x = x_ref[...]
    y_ref[...] = _dotT(x, wl_ref[...])
    z_ref[...] = _dotT(x, wr_ref[...]) + b_ref[...]


def _bn_relu(s_ref, degp_ref, z_ref, g_ref, be_ref):
    n = z_ref.shape[0]
    s = (s_ref[0] + s_ref[1])[:n]
    deg = (degp_ref[0] + degp_ref[1])[:n]
    h = s / jnp.maximum(deg, 1.0) + z_ref[...]
    mu = jnp.mean(h, axis=0, keepdims=True)
    ctr = h - mu
    var = jnp.mean(ctr * ctr, axis=0, keepdims=True)
    hn = g_ref[...] * ctr * lax.rsqrt(var + _EPS) + be_ref[...]
    return jnp.maximum(hn, 0.0)


def _mid_body(s_ref, degp_ref, z_ref, g_ref, be_ref, wl_ref, wr_ref,
              b_ref, y2_ref, z2_ref):
    h1 = _bn_relu(s_ref, degp_ref, z_ref, g_ref, be_ref)
    y2_ref[...] = _dotT(h1, wl_ref[...])
    z2_ref[...] = _dotT(h1, wr_ref[...]) + b_ref[...]


def _post_body(s_ref, degp_ref, z_ref, g_ref, be_ref, out_ref):
    out_ref[...] = _bn_relu(s_ref, degp_ref, z_ref, g_ref, be_ref)


def kernel(x, edge_index, Wl1, Wr1, b1, g1, be1, Wl2, Wr2, b2, g2, be2):
    N, D = x.shape
    Np = (N + 16 * 8 - 1) // (16 * 8) * (16 * 8)
    src1 = edge_index[0]
    dst1 = edge_index[1]
    zeros = jnp.zeros((Np, D), jnp.float32)
    ones = jnp.ones((_K, D), jnp.float32)
    f32 = jnp.float32
    sd = jax.ShapeDtypeStruct

    degp = _sc_degree(dst1, zeros, ones)

    y1, z1 = pl.pallas_call(
        _pre_body,
        out_shape=[sd((N, D), f32), sd((N, D), f32)],
    )(x, Wl1, Wr1, b1.reshape(1, D))

    s1 = _sc_scatter(y1, src1, dst1, zeros)

    y2, z2 = pl.pallas_call(
        _mid_body,
        out_shape=[sd((N, D), f32), sd((N, D), f32)],
    )(s1, degp, z1, g1.reshape(1, D), be1.reshape(1, D), Wl2, Wr2,
      b2.reshape(1, D))

    s2 = _sc_scatter(y2, src1, dst1, zeros)

    out = pl.pallas_call(
        _post_body,
        out_shape=sd((N, D), f32),
    )(s2, degp, z2, g2.reshape(1, D), be2.reshape(1, D))

    return out
